# Initial kernel scaffold; baseline (speedup 1.0000x reference)
import jax
import jax.numpy as jnp
from jax.experimental import pallas as pl


def kernel(x, edge_index, edge_attr, batch, atom_tables, bond_tables, eps, W1, b1, g1, bb1, W2, b2, Wout, bout):
    raise NotImplementedError("write your pallas kernel here")


# speedup vs baseline: 6.7721x; 6.7721x over previous
import functools

import jax
import jax.numpy as jnp
from jax import lax
from jax.experimental import pallas as pl
from jax.experimental.pallas import tpu as pltpu
from jax.experimental.pallas import tpu_sc as plsc

N = 10000
E = 320000
H = 128
L = 3
G = 128
C = 10
NA = 119
NB = 6

NC = 2
NS = 16
NW = NC * NS

_HIGH = lax.Precision.HIGHEST

NPAD = 10240
NPW = NPAD // NW
CHN = 64
NCHN = NPW // CHN
NCHN_TOT = NPAD // CHN


def _sc_mesh():
    return plsc.VectorSubcoreMesh(core_axis_name="c", subcore_axis_name="s",
                                  num_cores=NC, num_subcores=NS)


@functools.partial(
    pl.kernel,
    out_type=jax.ShapeDtypeStruct((NPAD, H), jnp.float32),
    mesh=_sc_mesh(),
    scratch_types=[
        pltpu.VMEM((9, CHN), jnp.int32),
        pltpu.VMEM((9, CHN, H), jnp.float32),
        pltpu.VMEM((CHN, H), jnp.float32),
        pltpu.SemaphoreType.DMA,
    ],
)
def _atom_encode_sc(xc_hbm, atab_hbm, out_hbm, idx_v, gbuf, acc, sem):
    c = lax.axis_index("c")
    s = lax.axis_index("s")
    wid = s * NC + c

    @pl.loop(0, NCHN)
    def _chunk(ch):
        chg = wid * NCHN + ch
        pltpu.sync_copy(xc_hbm.at[chg], idx_v)
        @pl.loop(0, CHN // 16)
        def _off(t):
            sl = pl.ds(t * 16, 16)
            for i in range(1, 9):
                idx_v[i, sl] = idx_v[i, sl] + (i * NA)
        cps = [pltpu.async_copy(atab_hbm.at[idx_v.at[i]], gbuf.at[i], sem)
               for i in range(9)]
        for cp in cps:
            cp.wait()

        @pl.loop(0, CHN)
        def _row(r):
            for k in range(8):
                sl = pl.ds(k * 16, 16)
                v = gbuf[0, r, sl]
                for i in range(1, 9):
                    v = v + gbuf[i, r, sl]
                acc[r, sl] = v

        pltpu.sync_copy(acc, out_hbm.at[pl.ds(chg * CHN, CHN)])


CHE = 80
EPW = E // NW
NCHE = EPW // CHE
NAGG = NPAD
RPT = NAGG // NS
ZR = 128


@functools.partial(
    pl.kernel,
    out_type=jax.ShapeDtypeStruct((NC, NAGG, H), jnp.float32),
    mesh=_sc_mesh(),
    scratch_types=[
        pltpu.VMEM((5, CHE), jnp.int32),
        pltpu.VMEM((CHE, H), jnp.float32),
        pltpu.VMEM((CHE, H), jnp.float32),
        pltpu.VMEM((ZR, H), jnp.float32),
        pltpu.VMEM_SHARED((NAGG, H), jnp.float32),
        pltpu.SemaphoreType.DMA,
        pltpu.SemaphoreType.DMA,
    ],
)
def _edge_pass_sc(h_hbm, einds_hbm, ctab_hbm, out_hbm,
                  idx5, hbuf, ebuf, zbuf, aggr, sem1, sem2):
    c = lax.axis_index("c")
    s = lax.axis_index("s")
    wid = c * NS + s

    zeros = jnp.zeros((16,), jnp.float32)

    @pl.loop(0, ZR)
    def _z(r):
        for k in range(8):
            zbuf[r, pl.ds(k * 16, 16)] = zeros

    row0 = s * RPT
    for j in range(RPT // ZR):
        pltpu.sync_copy(zbuf, aggr.at[pl.ds(row0 + j * ZR, ZR)])
    plsc.subcore_barrier()

    @pl.loop(0, NCHE)
    def _chunk(ch):
        chg = wid * NCHE + ch
        pltpu.sync_copy(einds_hbm.at[chg], idx5)
        @pl.loop(0, CHE // 16)
        def _comb(t):
            sl = pl.ds(t * 16, 16)
            idx5[1, sl] = (idx5[1, sl] * 6 + idx5[2, sl]) * 6 + idx5[3, sl]
        cp1 = pltpu.async_copy(h_hbm.at[idx5.at[0]], hbuf, sem1)
        cp2 = pltpu.async_copy(ctab_hbm.at[idx5.at[1]], ebuf, sem2)
        cp1.wait()
        cp2.wait()

        @pl.loop(0, CHE)
        def _row(r):
            for k in range(8):
                sl = pl.ds(k * 16, 16)
                hbuf[r, sl] = jnp.maximum(hbuf[r, sl] + ebuf[r, sl], 0.0)

        pltpu.sync_copy(hbuf, aggr.at[idx5.at[4]], add=True)

    plsc.subcore_barrier()
    for j in range(RPT // ZR):
        r0 = row0 + j * ZR
        pltpu.sync_copy(aggr.at[pl.ds(r0, ZR)], out_hbm.at[c, pl.ds(r0, ZR)])


BN_BLK = 2000
NBLK = N // BN_BLK


def _ctab_body(bt_ref, a_ref, b_ref, c_ref, out_ref):
    r = jnp.dot(a_ref[...], bt_ref[0, 0], precision=_HIGH,
                preferred_element_type=jnp.float32)
    r = r + jnp.dot(b_ref[...], bt_ref[0, 1], precision=_HIGH,
                    preferred_element_type=jnp.float32)
    r = r + jnp.dot(c_ref[...], bt_ref[0, 2], precision=_HIGH,
                    preferred_element_type=jnp.float32)
    out_ref[0] = r


_ctab_call = pl.pallas_call(
    _ctab_body,
    grid=(L,),
    in_specs=[
        pl.BlockSpec((1, 3, NB, H), lambda l: (l, 0, 0, 0)),
        pl.BlockSpec((216, NB), lambda l: (0, 0)),
        pl.BlockSpec((216, NB), lambda l: (0, 0)),
        pl.BlockSpec((216, NB), lambda l: (0, 0)),
    ],
    out_specs=pl.BlockSpec((1, 216, H), lambda l: (l, 0, 0)),
    out_shape=jax.ShapeDtypeStruct((L, 216, H), jnp.float32),
)


def _mlp1_body(h_ref, a_ref, eps_ref, w1_ref, b1_ref, z1_ref, ps_ref, pq_ref):
    zin = (1.0 + eps_ref[0, 0]) * h_ref[...] + a_ref[0] + a_ref[1]
    z1 = jnp.dot(zin, w1_ref[...], precision=_HIGH,
                 preferred_element_type=jnp.float32) + b1_ref[...]
    z1_ref[...] = z1
    ps_ref[0] = jnp.sum(z1, axis=0, keepdims=True)
    pq_ref[0] = jnp.sum(z1 * z1, axis=0, keepdims=True)


_mlp1_call = pl.pallas_call(
    _mlp1_body,
    grid=(NBLK,),
    in_specs=[
        pl.BlockSpec((BN_BLK, H), lambda i: (i, 0)),
        pl.BlockSpec((NC, BN_BLK, H), lambda i: (0, i, 0)),
        pl.BlockSpec((1, 1), lambda i: (0, 0)),
        pl.BlockSpec((H, 2 * H), lambda i: (0, 0)),
        pl.BlockSpec((1, 2 * H), lambda i: (0, 0)),
    ],
    out_specs=[
        pl.BlockSpec((BN_BLK, 2 * H), lambda i: (i, 0)),
        pl.BlockSpec((1, 1, 2 * H), lambda i: (i, 0, 0)),
        pl.BlockSpec((1, 1, 2 * H), lambda i: (i, 0, 0)),
    ],
    out_shape=[
        jax.ShapeDtypeStruct((N, 2 * H), jnp.float32),
        jax.ShapeDtypeStruct((NBLK, 1, 2 * H), jnp.float32),
        jax.ShapeDtypeStruct((NBLK, 1, 2 * H), jnp.float32),
    ],
)


def _mlp2_body(z1_ref, ps_ref, pq_ref, g1_ref, bb1_ref, w2_ref, b2_ref, out_ref):
    mu = jnp.sum(ps_ref[...], axis=0) * (1.0 / N)
    var = jnp.sum(pq_ref[...], axis=0) * (1.0 / N) - mu * mu
    z = z1_ref[...]
    zn = g1_ref[...] * (z - mu) / jnp.sqrt(var + 1e-5) + bb1_ref[...]
    zn = jnp.maximum(zn, 0.0)
    out_ref[...] = jnp.dot(zn, w2_ref[...], precision=_HIGH,
                           preferred_element_type=jnp.float32) + b2_ref[...]


_mlp2_call = pl.pallas_call(
    _mlp2_body,
    grid=(NBLK,),
    in_specs=[
        pl.BlockSpec((BN_BLK, 2 * H), lambda i: (i, 0)),
        pl.BlockSpec((NBLK, 1, 2 * H), lambda i: (0, 0, 0)),
        pl.BlockSpec((NBLK, 1, 2 * H), lambda i: (0, 0, 0)),
        pl.BlockSpec((1, 2 * H), lambda i: (0, 0)),
        pl.BlockSpec((1, 2 * H), lambda i: (0, 0)),
        pl.BlockSpec((2 * H, H), lambda i: (0, 0)),
        pl.BlockSpec((1, H), lambda i: (0, 0)),
    ],
    out_specs=pl.BlockSpec((BN_BLK, H), lambda i: (i, 0)),
    out_shape=jax.ShapeDtypeStruct((N, H), jnp.float32),
)


def _pool_body(h_ref, batch_ref, wout_ref, bout_ref, out_ref, acc_ref):
    i = pl.program_id(0)
    b = batch_ref[0, 0]
    oh = (b[:, None] == lax.broadcasted_iota(jnp.int32, (1, G), 1)
          ).astype(jnp.float32)
    part = lax.dot_general(oh, h_ref[...], (((0,), (0,)), ((), ())),
                           precision=_HIGH, preferred_element_type=jnp.float32)

    @pl.when(i == 0)
    def _():
        acc_ref[...] = part

    @pl.when(i > 0)
    def _():
        acc_ref[...] = acc_ref[...] + part

    @pl.when(i == NBLK - 1)
    def _():
        out_ref[...] = jnp.dot(acc_ref[...], wout_ref[...], precision=_HIGH,
                               preferred_element_type=jnp.float32) + bout_ref[...]


_pool_call = pl.pallas_call(
    _pool_body,
    grid=(NBLK,),
    in_specs=[
        pl.BlockSpec((BN_BLK, H), lambda i: (i, 0)),
        pl.BlockSpec((1, 1, BN_BLK), lambda i: (i, 0, 0)),
        pl.BlockSpec((H, C), lambda i: (0, 0)),
        pl.BlockSpec((1, C), lambda i: (0, 0)),
    ],
    out_specs=pl.BlockSpec((G, C), lambda i: (0, 0)),
    out_shape=jax.ShapeDtypeStruct((G, C), jnp.float32),
    scratch_shapes=[pltpu.VMEM((G, H), jnp.float32)],
)


def kernel(x, edge_index, edge_attr, batch, atom_tables, bond_tables, eps,
           W1, b1, g1, bb1, W2, b2, Wout, bout):
    x = x.astype(jnp.int32)
    ei = edge_index.astype(jnp.int32)
    ea = edge_attr.astype(jnp.int32)
    batch = batch.astype(jnp.int32)

    xt = jnp.pad(x.T, ((0, 0), (0, NPAD - N)))
    xc = xt.reshape(9, NCHN_TOT, CHN).transpose(1, 0, 2)
    atab = atom_tables.reshape(9 * NA, H)
    h = _atom_encode_sc(xc, atab)[:N]

    cid = jnp.arange(216)
    oh_a = jax.nn.one_hot(cid // 36, NB, dtype=jnp.float32)
    oh_b = jax.nn.one_hot((cid // 6) % NB, NB, dtype=jnp.float32)
    oh_c = jax.nn.one_hot(cid % NB, NB, dtype=jnp.float32)
    ctab = _ctab_call(bond_tables, oh_a, oh_b, oh_c)

    einds = jnp.stack([ei[0], ea[:, 0], ea[:, 1], ea[:, 2], ei[1]], axis=0)
    einds = einds.reshape(5, E // CHE, CHE).transpose(1, 0, 2)

    batch3 = batch.reshape(NBLK, 1, BN_BLK)

    for l in range(L):
        aggr2 = _edge_pass_sc(h, einds, ctab[l])[:, :N]
        z1, ps, pq = _mlp1_call(h, aggr2, eps[l].reshape(1, 1), W1[l],
                                b1[l].reshape(1, 2 * H))
        h = _mlp2_call(z1, ps, pq, g1[l].reshape(1, 2 * H),
                       bb1[l].reshape(1, 2 * H), W2[l], b2[l].reshape(1, H))

    return _pool_call(h, batch3, Wout, bout.reshape(1, C))


# speedup vs baseline: 9.5824x; 1.4150x over previous
import functools

import jax
import jax.numpy as jnp
from jax import lax
from jax.experimental import pallas as pl
from jax.experimental.pallas import tpu as pltpu
from jax.experimental.pallas import tpu_sc as plsc

N = 10000
E = 320000
H = 128
L = 3
G = 128
C = 10
NA = 119
NB = 6

NC = 2
NS = 16
NW = NC * NS

_HIGH = lax.Precision.HIGHEST

NPAD = 10240
NPW = NPAD // NW
CHN = 64
NCHN = NPW // CHN
NCHN_TOT = NPAD // CHN


def _sc_mesh():
    return plsc.VectorSubcoreMesh(core_axis_name="c", subcore_axis_name="s",
                                  num_cores=NC, num_subcores=NS)


@functools.partial(
    pl.kernel,
    out_type=jax.ShapeDtypeStruct((NPAD, H), jnp.float32),
    mesh=_sc_mesh(),
    scratch_types=[
        pltpu.VMEM((9, CHN), jnp.int32),
        pltpu.VMEM((9, CHN, H), jnp.float32),
        pltpu.VMEM((CHN, H), jnp.float32),
        pltpu.SemaphoreType.DMA,
    ],
)
def _atom_encode_sc(xc_hbm, atab_hbm, out_hbm, idx_v, gbuf, acc, sem):
    c = lax.axis_index("c")
    s = lax.axis_index("s")
    wid = s * NC + c

    @pl.loop(0, NCHN)
    def _chunk(ch):
        chg = wid * NCHN + ch
        pltpu.sync_copy(xc_hbm.at[chg], idx_v)
        @pl.loop(0, CHN // 16)
        def _off(t):
            sl = pl.ds(t * 16, 16)
            for i in range(1, 9):
                idx_v[i, sl] = idx_v[i, sl] + (i * NA)
        cps = [pltpu.async_copy(atab_hbm.at[idx_v.at[i]], gbuf.at[i], sem)
               for i in range(9)]
        for cp in cps:
            cp.wait()

        @pl.loop(0, CHN)
        def _row(r):
            for k in range(8):
                sl = pl.ds(k * 16, 16)
                v = gbuf[0, r, sl]
                for i in range(1, 9):
                    v = v + gbuf[i, r, sl]
                acc[r, sl] = v

        pltpu.sync_copy(acc, out_hbm.at[pl.ds(chg * CHN, CHN)])


CHE = 80
NCHT = E // CHE
NAGG = NPAD
RPT = NAGG // NS
ZR = CHE


@functools.partial(
    pl.kernel,
    out_type=jax.ShapeDtypeStruct((NC, NAGG, H), jnp.float32),
    mesh=_sc_mesh(),
    scratch_types=[
        pltpu.VMEM((2, 5, CHE), jnp.int32),
        pltpu.VMEM((2, CHE, H), jnp.float32),
        pltpu.VMEM((2, CHE, H), jnp.float32),
        pltpu.VMEM_SHARED((NAGG, H), jnp.float32),
        pltpu.SemaphoreType.DMA,
        pltpu.SemaphoreType.DMA,
        pltpu.SemaphoreType.DMA,
        pltpu.SemaphoreType.DMA,
    ],
)
def _edge_pass_sc(h_hbm, einds_hbm, ctab_hbm, out_hbm,
                  idx5, hbuf, ebuf, aggr, sh0, sh1, se0, se1):
    c = lax.axis_index("c")
    s = lax.axis_index("s")
    wid = c * NS + s
    semh = (sh0, sh1)
    seme = (se0, se1)

    zeros = jnp.zeros((16,), jnp.float32)

    @pl.loop(0, ZR)
    def _z(r):
        for k in range(8):
            ebuf[0, r, pl.ds(k * 16, 16)] = zeros

    row0 = s * RPT
    for j in range(RPT // ZR):
        pltpu.sync_copy(ebuf.at[0], aggr.at[pl.ds(row0 + j * ZR, ZR)])
    plsc.subcore_barrier()

    lo = wid * NCHT // NW
    nch = (wid + 1) * NCHT // NW - lo

    def _fetch(g, b):
        pltpu.sync_copy(einds_hbm.at[g], idx5.at[b])

        @pl.loop(0, CHE // 16)
        def _comb(t):
            sl = pl.ds(t * 16, 16)
            idx5[b, 1, sl] = ((idx5[b, 1, sl] * 6 + idx5[b, 2, sl]) * 6
                              + idx5[b, 3, sl])

        pltpu.async_copy(h_hbm.at[idx5.at[b, 0]], hbuf.at[b], semh[b])
        pltpu.async_copy(ctab_hbm.at[idx5.at[b, 1]], ebuf.at[b], seme[b])

    def _consume(b):
        pltpu.make_async_copy(h_hbm.at[idx5.at[b, 0]], hbuf.at[b],
                              semh[b]).wait()
        pltpu.make_async_copy(ctab_hbm.at[idx5.at[b, 1]], ebuf.at[b],
                              seme[b]).wait()

        @pl.loop(0, CHE)
        def _row(r):
            for k in range(8):
                sl = pl.ds(k * 16, 16)
                hbuf[b, r, sl] = jnp.maximum(hbuf[b, r, sl] + ebuf[b, r, sl],
                                             0.0)

        pltpu.sync_copy(hbuf.at[b], aggr.at[idx5.at[b, 4]], add=True)

    _fetch(lo, 0)

    @pl.loop(0, nch, step=2)
    def _pair(t):
        for b in range(2):
            g = lo + t + b

            @pl.when(g + 1 < lo + nch)
            def _():
                _fetch(g + 1, 1 - b)

            @pl.when(g < lo + nch)
            def _():
                _consume(b)

    plsc.subcore_barrier()
    for j in range(RPT // ZR):
        r0 = row0 + j * ZR
        pltpu.sync_copy(aggr.at[pl.ds(r0, ZR)], out_hbm.at[c, pl.ds(r0, ZR)])


BN_BLK = 2000
NBLK = N // BN_BLK


def _ctab_body(bt_ref, a_ref, b_ref, c_ref, out_ref):
    r = jnp.dot(a_ref[...], bt_ref[0, 0], precision=_HIGH,
                preferred_element_type=jnp.float32)
    r = r + jnp.dot(b_ref[...], bt_ref[0, 1], precision=_HIGH,
                    preferred_element_type=jnp.float32)
    r = r + jnp.dot(c_ref[...], bt_ref[0, 2], precision=_HIGH,
                    preferred_element_type=jnp.float32)
    out_ref[0] = r


_ctab_call = pl.pallas_call(
    _ctab_body,
    grid=(L,),
    in_specs=[
        pl.BlockSpec((1, 3, NB, H), lambda l: (l, 0, 0, 0)),
        pl.BlockSpec((216, NB), lambda l: (0, 0)),
        pl.BlockSpec((216, NB), lambda l: (0, 0)),
        pl.BlockSpec((216, NB), lambda l: (0, 0)),
    ],
    out_specs=pl.BlockSpec((1, 216, H), lambda l: (l, 0, 0)),
    out_shape=jax.ShapeDtypeStruct((L, 216, H), jnp.float32),
)


def _mlp1_body(h_ref, a_ref, eps_ref, w1_ref, b1_ref, z1_ref, ps_ref, pq_ref):
    zin = (1.0 + eps_ref[0, 0]) * h_ref[...] + a_ref[0] + a_ref[1]
    z1 = jnp.dot(zin, w1_ref[...], precision=_HIGH,
                 preferred_element_type=jnp.float32) + b1_ref[...]
    z1_ref[...] = z1
    ps_ref[0] = jnp.sum(z1, axis=0, keepdims=True)
    pq_ref[0] = jnp.sum(z1 * z1, axis=0, keepdims=True)


_mlp1_call = pl.pallas_call(
    _mlp1_body,
    grid=(NBLK,),
    in_specs=[
        pl.BlockSpec((BN_BLK, H), lambda i: (i, 0)),
        pl.BlockSpec((NC, BN_BLK, H), lambda i: (0, i, 0)),
        pl.BlockSpec((1, 1), lambda i: (0, 0)),
        pl.BlockSpec((H, 2 * H), lambda i: (0, 0)),
        pl.BlockSpec((1, 2 * H), lambda i: (0, 0)),
    ],
    out_specs=[
        pl.BlockSpec((BN_BLK, 2 * H), lambda i: (i, 0)),
        pl.BlockSpec((1, 1, 2 * H), lambda i: (i, 0, 0)),
        pl.BlockSpec((1, 1, 2 * H), lambda i: (i, 0, 0)),
    ],
    out_shape=[
        jax.ShapeDtypeStruct((N, 2 * H), jnp.float32),
        jax.ShapeDtypeStruct((NBLK, 1, 2 * H), jnp.float32),
        jax.ShapeDtypeStruct((NBLK, 1, 2 * H), jnp.float32),
    ],
)


def _mlp2_body(z1_ref, ps_ref, pq_ref, g1_ref, bb1_ref, w2_ref, b2_ref, out_ref):
    mu = jnp.sum(ps_ref[...], axis=0) * (1.0 / N)
    var = jnp.sum(pq_ref[...], axis=0) * (1.0 / N) - mu * mu
    z = z1_ref[...]
    zn = g1_ref[...] * (z - mu) / jnp.sqrt(var + 1e-5) + bb1_ref[...]
    zn = jnp.maximum(zn, 0.0)
    out_ref[...] = jnp.dot(zn, w2_ref[...], precision=_HIGH,
                           preferred_element_type=jnp.float32) + b2_ref[...]


_mlp2_call = pl.pallas_call(
    _mlp2_body,
    grid=(NBLK,),
    in_specs=[
        pl.BlockSpec((BN_BLK, 2 * H), lambda i: (i, 0)),
        pl.BlockSpec((NBLK, 1, 2 * H), lambda i: (0, 0, 0)),
        pl.BlockSpec((NBLK, 1, 2 * H), lambda i: (0, 0, 0)),
        pl.BlockSpec((1, 2 * H), lambda i: (0, 0)),
        pl.BlockSpec((1, 2 * H), lambda i: (0, 0)),
        pl.BlockSpec((2 * H, H), lambda i: (0, 0)),
        pl.BlockSpec((1, H), lambda i: (0, 0)),
    ],
    out_specs=pl.BlockSpec((BN_BLK, H), lambda i: (i, 0)),
    out_shape=jax.ShapeDtypeStruct((N, H), jnp.float32),
)


def _pool_body(h_ref, batch_ref, wout_ref, bout_ref, out_ref, acc_ref):
    i = pl.program_id(0)
    b = batch_ref[0, 0]
    oh = (b[:, None] == lax.broadcasted_iota(jnp.int32, (1, G), 1)
          ).astype(jnp.float32)
    part = lax.dot_general(oh, h_ref[...], (((0,), (0,)), ((), ())),
                           precision=_HIGH, preferred_element_type=jnp.float32)

    @pl.when(i == 0)
    def _():
        acc_ref[...] = part

    @pl.when(i > 0)
    def _():
        acc_ref[...] = acc_ref[...] + part

    @pl.when(i == NBLK - 1)
    def _():
        out_ref[...] = jnp.dot(acc_ref[...], wout_ref[...], precision=_HIGH,
                               preferred_element_type=jnp.float32) + bout_ref[...]


_pool_call = pl.pallas_call(
    _pool_body,
    grid=(NBLK,),
    in_specs=[
        pl.BlockSpec((BN_BLK, H), lambda i: (i, 0)),
        pl.BlockSpec((1, 1, BN_BLK), lambda i: (i, 0, 0)),
        pl.BlockSpec((H, C), lambda i: (0, 0)),
        pl.BlockSpec((1, C), lambda i: (0, 0)),
    ],
    out_specs=pl.BlockSpec((G, C), lambda i: (0, 0)),
    out_shape=jax.ShapeDtypeStruct((G, C), jnp.float32),
    scratch_shapes=[pltpu.VMEM((G, H), jnp.float32)],
)


def kernel(x, edge_index, edge_attr, batch, atom_tables, bond_tables, eps,
           W1, b1, g1, bb1, W2, b2, Wout, bout):
    x = x.astype(jnp.int32)
    ei = edge_index.astype(jnp.int32)
    ea = edge_attr.astype(jnp.int32)
    batch = batch.astype(jnp.int32)

    xt = jnp.pad(x.T, ((0, 0), (0, NPAD - N)))
    xc = xt.reshape(9, NCHN_TOT, CHN).transpose(1, 0, 2)
    atab = atom_tables.reshape(9 * NA, H)
    h = _atom_encode_sc(xc, atab)[:N]

    cid = jnp.arange(216)
    oh_a = jax.nn.one_hot(cid // 36, NB, dtype=jnp.float32)
    oh_b = jax.nn.one_hot((cid // 6) % NB, NB, dtype=jnp.float32)
    oh_c = jax.nn.one_hot(cid % NB, NB, dtype=jnp.float32)
    ctab = _ctab_call(bond_tables, oh_a, oh_b, oh_c)

    einds = jnp.stack([ei[0], ea[:, 0], ea[:, 1], ea[:, 2], ei[1]], axis=0)
    einds = einds.reshape(5, NCHT, CHE).transpose(1, 0, 2)

    batch3 = batch.reshape(NBLK, 1, BN_BLK)

    for l in range(L):
        aggr2 = _edge_pass_sc(h, einds, ctab[l])[:, :N]
        z1, ps, pq = _mlp1_call(h, aggr2, eps[l].reshape(1, 1), W1[l],
                                b1[l].reshape(1, 2 * H))
        h = _mlp2_call(z1, ps, pq, g1[l].reshape(1, 2 * H),
                       bb1[l].reshape(1, 2 * H), W2[l], b2[l].reshape(1, H))

    return _pool_call(h, batch3, Wout, bout.reshape(1, C))


# speedup vs baseline: 10.4009x; 1.0854x over previous
import functools

import jax
import jax.numpy as jnp
from jax import lax
from jax.experimental import pallas as pl
from jax.experimental.pallas import tpu as pltpu
from jax.experimental.pallas import tpu_sc as plsc

N = 10000
E = 320000
H = 128
L = 3
G = 128
C = 10
NA = 119
NB = 6

NC = 2
NS = 16
NW = NC * NS

_HIGH = lax.Precision.HIGHEST

NPAD = 10240
NPW = NPAD // NW
CHN = 64
NCHN = NPW // CHN
NCHN_TOT = NPAD // CHN


def _sc_mesh():
    return plsc.VectorSubcoreMesh(core_axis_name="c", subcore_axis_name="s",
                                  num_cores=NC, num_subcores=NS)


@functools.partial(
    pl.kernel,
    out_type=jax.ShapeDtypeStruct((NPAD, H), jnp.float32),
    mesh=_sc_mesh(),
    scratch_types=[
        pltpu.VMEM((9, CHN), jnp.int32),
        pltpu.VMEM((9, CHN, H), jnp.float32),
        pltpu.VMEM((CHN, H), jnp.float32),
        pltpu.SemaphoreType.DMA,
    ],
)
def _atom_encode_sc(xc_hbm, atab_hbm, out_hbm, idx_v, gbuf, acc, sem):
    c = lax.axis_index("c")
    s = lax.axis_index("s")
    wid = s * NC + c

    @pl.loop(0, NCHN)
    def _chunk(ch):
        chg = wid * NCHN + ch
        pltpu.sync_copy(xc_hbm.at[chg], idx_v)
        @pl.loop(0, CHN // 16)
        def _off(t):
            sl = pl.ds(t * 16, 16)
            for i in range(1, 9):
                idx_v[i, sl] = idx_v[i, sl] + (i * NA)
        cps = [pltpu.async_copy(atab_hbm.at[idx_v.at[i]], gbuf.at[i], sem)
               for i in range(9)]
        for cp in cps:
            cp.wait()

        @pl.loop(0, CHN)
        def _row(r):
            for k in range(8):
                sl = pl.ds(k * 16, 16)
                v = gbuf[0, r, sl]
                for i in range(1, 9):
                    v = v + gbuf[i, r, sl]
                acc[r, sl] = v

        pltpu.sync_copy(acc, out_hbm.at[pl.ds(chg * CHN, CHN)])


CHE = 80
NCHT = E // CHE
NAGG = NPAD
RPT = NAGG // NS
ZR = CHE


@functools.partial(
    pl.kernel,
    out_type=jax.ShapeDtypeStruct((NC, NAGG, H), jnp.float32),
    mesh=_sc_mesh(),
    scratch_types=[
        pltpu.VMEM((2, 5, CHE), jnp.int32),
        pltpu.VMEM((2, CHE, H), jnp.float32),
        pltpu.VMEM((2, CHE, H), jnp.float32),
        pltpu.VMEM_SHARED((NAGG, H), jnp.float32),
        pltpu.VMEM_SHARED((216, H), jnp.float32),
        pltpu.SemaphoreType.DMA,
        pltpu.SemaphoreType.DMA,
        pltpu.SemaphoreType.DMA,
        pltpu.SemaphoreType.DMA,
    ],
)
def _edge_pass_sc(h_hbm, einds_hbm, ctab_hbm, out_hbm,
                  idx5, hbuf, ebuf, aggr, ctab_sh, sh0, sh1, se0, se1):
    c = lax.axis_index("c")
    s = lax.axis_index("s")
    wid = c * NS + s
    semh = (sh0, sh1)
    seme = (se0, se1)

    zeros = jnp.zeros((16,), jnp.float32)

    @pl.when(s == 0)
    def _stage():
        pltpu.sync_copy(ctab_hbm, ctab_sh)

    @pl.loop(0, ZR)
    def _z(r):
        for k in range(8):
            ebuf[0, r, pl.ds(k * 16, 16)] = zeros

    row0 = s * RPT
    for j in range(RPT // ZR):
        pltpu.sync_copy(ebuf.at[0], aggr.at[pl.ds(row0 + j * ZR, ZR)])
    plsc.subcore_barrier()

    lo = wid * NCHT // NW
    nch = (wid + 1) * NCHT // NW - lo

    def _fetch(g, b):
        pltpu.sync_copy(einds_hbm.at[g], idx5.at[b])

        @pl.loop(0, CHE // 16)
        def _comb(t):
            sl = pl.ds(t * 16, 16)
            idx5[b, 1, sl] = ((idx5[b, 1, sl] * 6 + idx5[b, 2, sl]) * 6
                              + idx5[b, 3, sl])

        pltpu.async_copy(h_hbm.at[idx5.at[b, 0]], hbuf.at[b], semh[b])
        pltpu.async_copy(ctab_sh.at[idx5.at[b, 1]], ebuf.at[b], seme[b])

    def _consume(b):
        pltpu.make_async_copy(h_hbm.at[idx5.at[b, 0]], hbuf.at[b],
                              semh[b]).wait()
        pltpu.make_async_copy(ctab_sh.at[idx5.at[b, 1]], ebuf.at[b],
                              seme[b]).wait()

        @pl.loop(0, CHE)
        def _row(r):
            for k in range(8):
                sl = pl.ds(k * 16, 16)
                hbuf[b, r, sl] = jnp.maximum(hbuf[b, r, sl] + ebuf[b, r, sl],
                                             0.0)

        pltpu.sync_copy(hbuf.at[b], aggr.at[idx5.at[b, 4]], add=True)

    _fetch(lo, 0)

    @pl.loop(0, nch, step=2)
    def _pair(t):
        for b in range(2):
            g = lo + t + b

            @pl.when(g + 1 < lo + nch)
            def _():
                _fetch(g + 1, 1 - b)

            @pl.when(g < lo + nch)
            def _():
                _consume(b)

    plsc.subcore_barrier()
    for j in range(RPT // ZR):
        r0 = row0 + j * ZR
        pltpu.sync_copy(aggr.at[pl.ds(r0, ZR)], out_hbm.at[c, pl.ds(r0, ZR)])


BN_BLK = 2000
NBLK = N // BN_BLK


def _ctab_body(bt_ref, a_ref, b_ref, c_ref, out_ref):
    r = jnp.dot(a_ref[...], bt_ref[0, 0], precision=_HIGH,
                preferred_element_type=jnp.float32)
    r = r + jnp.dot(b_ref[...], bt_ref[0, 1], precision=_HIGH,
                    preferred_element_type=jnp.float32)
    r = r + jnp.dot(c_ref[...], bt_ref[0, 2], precision=_HIGH,
                    preferred_element_type=jnp.float32)
    out_ref[0] = r


_ctab_call = pl.pallas_call(
    _ctab_body,
    grid=(L,),
    in_specs=[
        pl.BlockSpec((1, 3, NB, H), lambda l: (l, 0, 0, 0)),
        pl.BlockSpec((216, NB), lambda l: (0, 0)),
        pl.BlockSpec((216, NB), lambda l: (0, 0)),
        pl.BlockSpec((216, NB), lambda l: (0, 0)),
    ],
    out_specs=pl.BlockSpec((1, 216, H), lambda l: (l, 0, 0)),
    out_shape=jax.ShapeDtypeStruct((L, 216, H), jnp.float32),
)


def _mlp1_body(h_ref, a_ref, eps_ref, w1_ref, b1_ref, z1_ref, ps_ref, pq_ref):
    zin = (1.0 + eps_ref[0, 0]) * h_ref[...] + a_ref[0] + a_ref[1]
    z1 = jnp.dot(zin, w1_ref[...], precision=_HIGH,
                 preferred_element_type=jnp.float32) + b1_ref[...]
    z1_ref[...] = z1
    ps_ref[0] = jnp.sum(z1, axis=0, keepdims=True)
    pq_ref[0] = jnp.sum(z1 * z1, axis=0, keepdims=True)


_mlp1_call = pl.pallas_call(
    _mlp1_body,
    grid=(NBLK,),
    in_specs=[
        pl.BlockSpec((BN_BLK, H), lambda i: (i, 0)),
        pl.BlockSpec((NC, BN_BLK, H), lambda i: (0, i, 0)),
        pl.BlockSpec((1, 1), lambda i: (0, 0)),
        pl.BlockSpec((H, 2 * H), lambda i: (0, 0)),
        pl.BlockSpec((1, 2 * H), lambda i: (0, 0)),
    ],
    out_specs=[
        pl.BlockSpec((BN_BLK, 2 * H), lambda i: (i, 0)),
        pl.BlockSpec((1, 1, 2 * H), lambda i: (i, 0, 0)),
        pl.BlockSpec((1, 1, 2 * H), lambda i: (i, 0, 0)),
    ],
    out_shape=[
        jax.ShapeDtypeStruct((N, 2 * H), jnp.float32),
        jax.ShapeDtypeStruct((NBLK, 1, 2 * H), jnp.float32),
        jax.ShapeDtypeStruct((NBLK, 1, 2 * H), jnp.float32),
    ],
)


def _mlp2_body(z1_ref, ps_ref, pq_ref, g1_ref, bb1_ref, w2_ref, b2_ref, out_ref):
    mu = jnp.sum(ps_ref[...], axis=0) * (1.0 / N)
    var = jnp.sum(pq_ref[...], axis=0) * (1.0 / N) - mu * mu
    z = z1_ref[...]
    zn = g1_ref[...] * (z - mu) / jnp.sqrt(var + 1e-5) + bb1_ref[...]
    zn = jnp.maximum(zn, 0.0)
    out_ref[...] = jnp.dot(zn, w2_ref[...], precision=_HIGH,
                           preferred_element_type=jnp.float32) + b2_ref[...]


_mlp2_call = pl.pallas_call(
    _mlp2_body,
    grid=(NBLK,),
    in_specs=[
        pl.BlockSpec((BN_BLK, 2 * H), lambda i: (i, 0)),
        pl.BlockSpec((NBLK, 1, 2 * H), lambda i: (0, 0, 0)),
        pl.BlockSpec((NBLK, 1, 2 * H), lambda i: (0, 0, 0)),
        pl.BlockSpec((1, 2 * H), lambda i: (0, 0)),
        pl.BlockSpec((1, 2 * H), lambda i: (0, 0)),
        pl.BlockSpec((2 * H, H), lambda i: (0, 0)),
        pl.BlockSpec((1, H), lambda i: (0, 0)),
    ],
    out_specs=pl.BlockSpec((BN_BLK, H), lambda i: (i, 0)),
    out_shape=jax.ShapeDtypeStruct((N, H), jnp.float32),
)


def _pool_body(h_ref, batch_ref, wout_ref, bout_ref, out_ref, acc_ref):
    i = pl.program_id(0)
    b = batch_ref[0, 0]
    oh = (b[:, None] == lax.broadcasted_iota(jnp.int32, (1, G), 1)
          ).astype(jnp.float32)
    part = lax.dot_general(oh, h_ref[...], (((0,), (0,)), ((), ())),
                           precision=_HIGH, preferred_element_type=jnp.float32)

    @pl.when(i == 0)
    def _():
        acc_ref[...] = part

    @pl.when(i > 0)
    def _():
        acc_ref[...] = acc_ref[...] + part

    @pl.when(i == NBLK - 1)
    def _():
        out_ref[...] = jnp.dot(acc_ref[...], wout_ref[...], precision=_HIGH,
                               preferred_element_type=jnp.float32) + bout_ref[...]


_pool_call = pl.pallas_call(
    _pool_body,
    grid=(NBLK,),
    in_specs=[
        pl.BlockSpec((BN_BLK, H), lambda i: (i, 0)),
        pl.BlockSpec((1, 1, BN_BLK), lambda i: (i, 0, 0)),
        pl.BlockSpec((H, C), lambda i: (0, 0)),
        pl.BlockSpec((1, C), lambda i: (0, 0)),
    ],
    out_specs=pl.BlockSpec((G, C), lambda i: (0, 0)),
    out_shape=jax.ShapeDtypeStruct((G, C), jnp.float32),
    scratch_shapes=[pltpu.VMEM((G, H), jnp.float32)],
)


def kernel(x, edge_index, edge_attr, batch, atom_tables, bond_tables, eps,
           W1, b1, g1, bb1, W2, b2, Wout, bout):
    x = x.astype(jnp.int32)
    ei = edge_index.astype(jnp.int32)
    ea = edge_attr.astype(jnp.int32)
    batch = batch.astype(jnp.int32)

    xt = jnp.pad(x.T, ((0, 0), (0, NPAD - N)))
    xc = xt.reshape(9, NCHN_TOT, CHN).transpose(1, 0, 2)
    atab = atom_tables.reshape(9 * NA, H)
    h = _atom_encode_sc(xc, atab)[:N]

    cid = jnp.arange(216)
    oh_a = jax.nn.one_hot(cid // 36, NB, dtype=jnp.float32)
    oh_b = jax.nn.one_hot((cid // 6) % NB, NB, dtype=jnp.float32)
    oh_c = jax.nn.one_hot(cid % NB, NB, dtype=jnp.float32)
    ctab = _ctab_call(bond_tables, oh_a, oh_b, oh_c)

    einds = jnp.stack([ei[0], ea[:, 0], ea[:, 1], ea[:, 2], ei[1]], axis=0)
    einds = einds.reshape(5, NCHT, CHE).transpose(1, 0, 2)

    batch3 = batch.reshape(NBLK, 1, BN_BLK)

    for l in range(L):
        aggr2 = _edge_pass_sc(h, einds, ctab[l])[:, :N]
        z1, ps, pq = _mlp1_call(h, aggr2, eps[l].reshape(1, 1), W1[l],
                                b1[l].reshape(1, 2 * H))
        h = _mlp2_call(z1, ps, pq, g1[l].reshape(1, 2 * H),
                       bb1[l].reshape(1, 2 * H), W2[l], b2[l].reshape(1, H))

    return _pool_call(h, batch3, Wout, bout.reshape(1, C))


# speedup vs baseline: 11.8058x; 1.1351x over previous
import functools

import jax
import jax.numpy as jnp
from jax import lax
from jax.experimental import pallas as pl
from jax.experimental.pallas import tpu as pltpu
from jax.experimental.pallas import tpu_sc as plsc

N = 10000
E = 320000
H = 128
L = 3
G = 128
C = 10
NA = 119
NB = 6

NC = 2
NS = 16
NW = NC * NS

_HIGH = lax.Precision.HIGHEST

NPAD = 10240
NPW = NPAD // NW
CHN = 64
NCHN = NPW // CHN
NCHN_TOT = NPAD // CHN


def _sc_mesh():
    return plsc.VectorSubcoreMesh(core_axis_name="c", subcore_axis_name="s",
                                  num_cores=NC, num_subcores=NS)


@functools.partial(
    pl.kernel,
    out_type=jax.ShapeDtypeStruct((NPAD, H), jnp.float32),
    mesh=_sc_mesh(),
    scratch_types=[
        pltpu.VMEM((9, CHN), jnp.int32),
        pltpu.VMEM((9, CHN, H), jnp.float32),
        pltpu.VMEM((CHN, H), jnp.float32),
        pltpu.SemaphoreType.DMA,
    ],
)
def _atom_encode_sc(xc_hbm, atab_hbm, out_hbm, idx_v, gbuf, acc, sem):
    c = lax.axis_index("c")
    s = lax.axis_index("s")
    wid = s * NC + c

    @pl.loop(0, NCHN)
    def _chunk(ch):
        chg = wid * NCHN + ch
        pltpu.sync_copy(xc_hbm.at[chg], idx_v)
        @pl.loop(0, CHN // 16)
        def _off(t):
            sl = pl.ds(t * 16, 16)
            for i in range(1, 9):
                idx_v[i, sl] = idx_v[i, sl] + (i * NA)
        cps = [pltpu.async_copy(atab_hbm.at[idx_v.at[i]], gbuf.at[i], sem)
               for i in range(9)]
        for cp in cps:
            cp.wait()

        @pl.loop(0, CHN)
        def _row(r):
            for k in range(8):
                sl = pl.ds(k * 16, 16)
                v = gbuf[0, r, sl]
                for i in range(1, 9):
                    v = v + gbuf[i, r, sl]
                acc[r, sl] = v

        pltpu.sync_copy(acc, out_hbm.at[pl.ds(chg * CHN, CHN)])


CHE = 80
NCHT = E // CHE
NAGG = NPAD
RPT = NAGG // NS
ZR = CHE


@functools.partial(
    pl.kernel,
    out_type=jax.ShapeDtypeStruct((NC, NAGG, H), jnp.float32),
    mesh=_sc_mesh(),
    scratch_types=[
        pltpu.VMEM((3, 5, CHE), jnp.int32),
        pltpu.VMEM((2, CHE, H), jnp.float32),
        pltpu.VMEM((2, CHE, H), jnp.float32),
        pltpu.VMEM_SHARED((NAGG, H), jnp.float32),
        pltpu.VMEM_SHARED((216, H), jnp.float32),
        pltpu.SemaphoreType.DMA,
        pltpu.SemaphoreType.DMA,
        pltpu.SemaphoreType.DMA,
        pltpu.SemaphoreType.DMA,
        pltpu.SemaphoreType.DMA,
        pltpu.SemaphoreType.DMA,
        pltpu.SemaphoreType.DMA,
        pltpu.SemaphoreType.DMA,
        pltpu.SemaphoreType.DMA,
    ],
)
def _edge_pass_sc(h_hbm, einds_hbm, ctab_hbm, out_hbm,
                  idx5, hbuf, ebuf, aggr, ctab_sh,
                  sh0, sh1, se0, se1, si0, si1, si2, ss0, ss1):
    c = lax.axis_index("c")
    s = lax.axis_index("s")
    wid = c * NS + s
    semh = (sh0, sh1)
    seme = (se0, se1)
    semi = (si0, si1, si2)
    semsc = (ss0, ss1)

    zeros = jnp.zeros((16,), jnp.float32)

    @pl.when(s == 0)
    def _stage():
        pltpu.sync_copy(ctab_hbm, ctab_sh)

    @pl.loop(0, ZR)
    def _z(r):
        for k in range(8):
            ebuf[0, r, pl.ds(k * 16, 16)] = zeros

    row0 = s * RPT
    for j in range(RPT // ZR):
        pltpu.sync_copy(ebuf.at[0], aggr.at[pl.ds(row0 + j * ZR, ZR)])
    plsc.subcore_barrier()

    nch = NCHT // NW
    lo = wid * nch

    def _start_idx(g, sl_):
        pltpu.async_copy(einds_hbm.at[lo + g], idx5.at[sl_], semi[sl_])

    def _wait_idx(g, sl_):
        pltpu.make_async_copy(einds_hbm.at[lo + g], idx5.at[sl_],
                              semi[sl_]).wait()

    def _start_gathers(sl_, b):
        @pl.loop(0, CHE // 16)
        def _comb(t):
            s2 = pl.ds(t * 16, 16)
            idx5[sl_, 1, s2] = ((idx5[sl_, 1, s2] * 6 + idx5[sl_, 2, s2]) * 6
                                + idx5[sl_, 3, s2])

        pltpu.async_copy(h_hbm.at[idx5.at[sl_, 0]], hbuf.at[b], semh[b])
        pltpu.async_copy(ctab_sh.at[idx5.at[sl_, 1]], ebuf.at[b], seme[b])

    def _compute_scatter(sl_, b):
        pltpu.make_async_copy(h_hbm.at[idx5.at[sl_, 0]], hbuf.at[b],
                              semh[b]).wait()
        pltpu.make_async_copy(ctab_sh.at[idx5.at[sl_, 1]], ebuf.at[b],
                              seme[b]).wait()

        @pl.loop(0, CHE)
        def _row(r):
            for k in range(8):
                s2 = pl.ds(k * 16, 16)
                hbuf[b, r, s2] = jnp.maximum(hbuf[b, r, s2] + ebuf[b, r, s2],
                                             0.0)

        pltpu.async_copy(hbuf.at[b], aggr.at[idx5.at[sl_, 4]], semsc[b],
                         add=True)

    def _wait_scatter(sl_, b):
        pltpu.make_async_copy(hbuf.at[b], aggr.at[idx5.at[sl_, 4]],
                              semsc[b]).wait()

    _start_idx(0, 0)
    _start_idx(1, 1)
    _wait_idx(0, 0)
    _start_gathers(0, 0)

    @pl.loop(0, nch + 1, step=6)
    def _body(t):
        for ph in range(6):
            g = t + ph
            b = ph % 2
            sl_g = ph % 3
            sl_n = (ph + 1) % 3
            sl_p = (ph + 2) % 3

            @pl.when(jnp.logical_and(g >= 1, g <= nch))
            def _():
                _wait_scatter(sl_p, 1 - b)

            @pl.when(g + 2 < nch)
            def _():
                _start_idx(g + 2, sl_p)

            @pl.when(g + 1 < nch)
            def _():
                _wait_idx(g + 1, sl_n)
                _start_gathers(sl_n, 1 - b)

            @pl.when(g < nch)
            def _():
                _compute_scatter(sl_g, b)

    plsc.subcore_barrier()
    for j in range(RPT // ZR):
        r0 = row0 + j * ZR
        pltpu.sync_copy(aggr.at[pl.ds(r0, ZR)], out_hbm.at[c, pl.ds(r0, ZR)])


BN_BLK = 2000
NBLK = N // BN_BLK


def _ctab_body(bt_ref, a_ref, b_ref, c_ref, out_ref):
    r = jnp.dot(a_ref[...], bt_ref[0, 0], precision=_HIGH,
                preferred_element_type=jnp.float32)
    r = r + jnp.dot(b_ref[...], bt_ref[0, 1], precision=_HIGH,
                    preferred_element_type=jnp.float32)
    r = r + jnp.dot(c_ref[...], bt_ref[0, 2], precision=_HIGH,
                    preferred_element_type=jnp.float32)
    out_ref[0] = r


_ctab_call = pl.pallas_call(
    _ctab_body,
    grid=(L,),
    in_specs=[
        pl.BlockSpec((1, 3, NB, H), lambda l: (l, 0, 0, 0)),
        pl.BlockSpec((216, NB), lambda l: (0, 0)),
        pl.BlockSpec((216, NB), lambda l: (0, 0)),
        pl.BlockSpec((216, NB), lambda l: (0, 0)),
    ],
    out_specs=pl.BlockSpec((1, 216, H), lambda l: (l, 0, 0)),
    out_shape=jax.ShapeDtypeStruct((L, 216, H), jnp.float32),
)


def _mlp1_body(h_ref, a_ref, eps_ref, w1_ref, b1_ref, z1_ref, ps_ref, pq_ref):
    zin = (1.0 + eps_ref[0, 0]) * h_ref[...] + a_ref[0] + a_ref[1]
    z1 = jnp.dot(zin, w1_ref[...], precision=_HIGH,
                 preferred_element_type=jnp.float32) + b1_ref[...]
    z1_ref[...] = z1
    ps_ref[0] = jnp.sum(z1, axis=0, keepdims=True)
    pq_ref[0] = jnp.sum(z1 * z1, axis=0, keepdims=True)


_mlp1_call = pl.pallas_call(
    _mlp1_body,
    grid=(NBLK,),
    in_specs=[
        pl.BlockSpec((BN_BLK, H), lambda i: (i, 0)),
        pl.BlockSpec((NC, BN_BLK, H), lambda i: (0, i, 0)),
        pl.BlockSpec((1, 1), lambda i: (0, 0)),
        pl.BlockSpec((H, 2 * H), lambda i: (0, 0)),
        pl.BlockSpec((1, 2 * H), lambda i: (0, 0)),
    ],
    out_specs=[
        pl.BlockSpec((BN_BLK, 2 * H), lambda i: (i, 0)),
        pl.BlockSpec((1, 1, 2 * H), lambda i: (i, 0, 0)),
        pl.BlockSpec((1, 1, 2 * H), lambda i: (i, 0, 0)),
    ],
    out_shape=[
        jax.ShapeDtypeStruct((N, 2 * H), jnp.float32),
        jax.ShapeDtypeStruct((NBLK, 1, 2 * H), jnp.float32),
        jax.ShapeDtypeStruct((NBLK, 1, 2 * H), jnp.float32),
    ],
)


def _mlp2_body(z1_ref, ps_ref, pq_ref, g1_ref, bb1_ref, w2_ref, b2_ref, out_ref):
    mu = jnp.sum(ps_ref[...], axis=0) * (1.0 / N)
    var = jnp.sum(pq_ref[...], axis=0) * (1.0 / N) - mu * mu
    z = z1_ref[...]
    zn = g1_ref[...] * (z - mu) / jnp.sqrt(var + 1e-5) + bb1_ref[...]
    zn = jnp.maximum(zn, 0.0)
    out_ref[...] = jnp.dot(zn, w2_ref[...], precision=_HIGH,
                           preferred_element_type=jnp.float32) + b2_ref[...]


_mlp2_call = pl.pallas_call(
    _mlp2_body,
    grid=(NBLK,),
    in_specs=[
        pl.BlockSpec((BN_BLK, 2 * H), lambda i: (i, 0)),
        pl.BlockSpec((NBLK, 1, 2 * H), lambda i: (0, 0, 0)),
        pl.BlockSpec((NBLK, 1, 2 * H), lambda i: (0, 0, 0)),
        pl.BlockSpec((1, 2 * H), lambda i: (0, 0)),
        pl.BlockSpec((1, 2 * H), lambda i: (0, 0)),
        pl.BlockSpec((2 * H, H), lambda i: (0, 0)),
        pl.BlockSpec((1, H), lambda i: (0, 0)),
    ],
    out_specs=pl.BlockSpec((BN_BLK, H), lambda i: (i, 0)),
    out_shape=jax.ShapeDtypeStruct((N, H), jnp.float32),
)


def _pool_body(h_ref, batch_ref, wout_ref, bout_ref, out_ref, acc_ref):
    i = pl.program_id(0)
    b = batch_ref[0, 0]
    oh = (b[:, None] == lax.broadcasted_iota(jnp.int32, (1, G), 1)
          ).astype(jnp.float32)
    part = lax.dot_general(oh, h_ref[...], (((0,), (0,)), ((), ())),
                           precision=_HIGH, preferred_element_type=jnp.float32)

    @pl.when(i == 0)
    def _():
        acc_ref[...] = part

    @pl.when(i > 0)
    def _():
        acc_ref[...] = acc_ref[...] + part

    @pl.when(i == NBLK - 1)
    def _():
        out_ref[...] = jnp.dot(acc_ref[...], wout_ref[...], precision=_HIGH,
                               preferred_element_type=jnp.float32) + bout_ref[...]


_pool_call = pl.pallas_call(
    _pool_body,
    grid=(NBLK,),
    in_specs=[
        pl.BlockSpec((BN_BLK, H), lambda i: (i, 0)),
        pl.BlockSpec((1, 1, BN_BLK), lambda i: (i, 0, 0)),
        pl.BlockSpec((H, C), lambda i: (0, 0)),
        pl.BlockSpec((1, C), lambda i: (0, 0)),
    ],
    out_specs=pl.BlockSpec((G, C), lambda i: (0, 0)),
    out_shape=jax.ShapeDtypeStruct((G, C), jnp.float32),
    scratch_shapes=[pltpu.VMEM((G, H), jnp.float32)],
)


def kernel(x, edge_index, edge_attr, batch, atom_tables, bond_tables, eps,
           W1, b1, g1, bb1, W2, b2, Wout, bout):
    x = x.astype(jnp.int32)
    ei = edge_index.astype(jnp.int32)
    ea = edge_attr.astype(jnp.int32)
    batch = batch.astype(jnp.int32)

    xt = jnp.pad(x.T, ((0, 0), (0, NPAD - N)))
    xc = xt.reshape(9, NCHN_TOT, CHN).transpose(1, 0, 2)
    atab = atom_tables.reshape(9 * NA, H)
    h = _atom_encode_sc(xc, atab)[:N]

    cid = jnp.arange(216)
    oh_a = jax.nn.one_hot(cid // 36, NB, dtype=jnp.float32)
    oh_b = jax.nn.one_hot((cid // 6) % NB, NB, dtype=jnp.float32)
    oh_c = jax.nn.one_hot(cid % NB, NB, dtype=jnp.float32)
    ctab = _ctab_call(bond_tables, oh_a, oh_b, oh_c)

    einds = jnp.stack([ei[0], ea[:, 0], ea[:, 1], ea[:, 2], ei[1]], axis=0)
    einds = einds.reshape(5, NCHT, CHE).transpose(1, 0, 2)

    batch3 = batch.reshape(NBLK, 1, BN_BLK)

    for l in range(L):
        aggr2 = _edge_pass_sc(h, einds, ctab[l])
        z1, ps, pq = _mlp1_call(h, aggr2, eps[l].reshape(1, 1), W1[l],
                                b1[l].reshape(1, 2 * H))
        h = _mlp2_call(z1, ps, pq, g1[l].reshape(1, 2 * H),
                       bb1[l].reshape(1, 2 * H), W2[l], b2[l].reshape(1, H))

    return _pool_call(h, batch3, Wout, bout.reshape(1, C))


# speedup vs baseline: 12.3171x; 1.0433x over previous
import functools

import jax
import jax.numpy as jnp
from jax import lax
from jax.experimental import pallas as pl
from jax.experimental.pallas import tpu as pltpu
from jax.experimental.pallas import tpu_sc as plsc

N = 10000
E = 320000
H = 128
L = 3
G = 128
C = 10
NA = 119
NB = 6

NC = 2
NS = 16
NW = NC * NS

_HIGH = lax.Precision.HIGHEST

NPAD = 10240
NPW = NPAD // NW
CHN = 64
NCHN = NPW // CHN
NCHN_TOT = NPAD // CHN


def _sc_mesh():
    return plsc.VectorSubcoreMesh(core_axis_name="c", subcore_axis_name="s",
                                  num_cores=NC, num_subcores=NS)


@functools.partial(
    pl.kernel,
    out_type=jax.ShapeDtypeStruct((NPAD, H), jnp.float32),
    mesh=_sc_mesh(),
    scratch_types=[
        pltpu.VMEM((9, CHN), jnp.int32),
        pltpu.VMEM((9, CHN, H), jnp.float32),
        pltpu.VMEM((CHN, H), jnp.float32),
        pltpu.SemaphoreType.DMA,
    ],
)
def _atom_encode_sc(xc_hbm, atab_hbm, out_hbm, idx_v, gbuf, acc, sem):
    c = lax.axis_index("c")
    s = lax.axis_index("s")
    wid = s * NC + c

    @pl.loop(0, NCHN)
    def _chunk(ch):
        chg = wid * NCHN + ch
        pltpu.sync_copy(xc_hbm.at[chg], idx_v)
        for t in range(CHN // 16):
            sl = pl.ds(t * 16, 16)
            for i in range(1, 9):
                idx_v[i, sl] = idx_v[i, sl] + (i * NA)
        cps = [pltpu.async_copy(atab_hbm.at[idx_v.at[i]], gbuf.at[i], sem)
               for i in range(9)]
        for cp in cps:
            cp.wait()

        @plsc.parallel_loop(0, CHN, 1, unroll=2)
        def _row(r):
            for k in range(8):
                sl = pl.ds(k * 16, 16)
                v = gbuf[0, r, sl]
                for i in range(1, 9):
                    v = v + gbuf[i, r, sl]
                acc[r, sl] = v

        pltpu.sync_copy(acc, out_hbm.at[pl.ds(chg * CHN, CHN)])


CHE = 80
NCHT = E // CHE
NAGG = NPAD
RPT = NAGG // NS
ZR = CHE


@functools.partial(
    pl.kernel,
    out_type=jax.ShapeDtypeStruct((NC, NAGG, H), jnp.float32),
    mesh=_sc_mesh(),
    scratch_types=[
        pltpu.VMEM((3, 5, CHE), jnp.int32),
        pltpu.VMEM((2, CHE, H), jnp.float32),
        pltpu.VMEM((2, CHE, H), jnp.float32),
        pltpu.VMEM_SHARED((NAGG, H), jnp.float32),
        pltpu.VMEM_SHARED((216, H), jnp.float32),
        pltpu.SemaphoreType.DMA,
        pltpu.SemaphoreType.DMA,
        pltpu.SemaphoreType.DMA,
        pltpu.SemaphoreType.DMA,
        pltpu.SemaphoreType.DMA,
        pltpu.SemaphoreType.DMA,
        pltpu.SemaphoreType.DMA,
        pltpu.SemaphoreType.DMA,
        pltpu.SemaphoreType.DMA,
    ],
)
def _edge_pass_sc(h_hbm, einds_hbm, ctab_hbm, out_hbm,
                  idx5, hbuf, ebuf, aggr, ctab_sh,
                  sh0, sh1, se0, se1, si0, si1, si2, ss0, ss1):
    c = lax.axis_index("c")
    s = lax.axis_index("s")
    wid = c * NS + s
    semh = (sh0, sh1)
    seme = (se0, se1)
    semi = (si0, si1, si2)
    semsc = (ss0, ss1)

    zeros = jnp.zeros((16,), jnp.float32)

    @pl.when(s == 0)
    def _stage():
        pltpu.sync_copy(ctab_hbm, ctab_sh)

    @plsc.parallel_loop(0, ZR, 1, unroll=4)
    def _z(r):
        for k in range(8):
            ebuf[0, r, pl.ds(k * 16, 16)] = zeros

    row0 = s * RPT
    for j in range(RPT // ZR):
        pltpu.sync_copy(ebuf.at[0], aggr.at[pl.ds(row0 + j * ZR, ZR)])
    plsc.subcore_barrier()

    nch = NCHT // NW
    lo = wid * nch

    def _start_idx(g, sl_):
        pltpu.async_copy(einds_hbm.at[lo + g], idx5.at[sl_], semi[sl_])

    def _wait_idx(g, sl_):
        pltpu.make_async_copy(einds_hbm.at[lo + g], idx5.at[sl_],
                              semi[sl_]).wait()

    def _start_gathers(sl_, b):
        for t in range(CHE // 16):
            s2 = pl.ds(t * 16, 16)
            idx5[sl_, 1, s2] = ((idx5[sl_, 1, s2] * 6 + idx5[sl_, 2, s2]) * 6
                                + idx5[sl_, 3, s2])

        pltpu.async_copy(h_hbm.at[idx5.at[sl_, 0]], hbuf.at[b], semh[b])
        pltpu.async_copy(ctab_sh.at[idx5.at[sl_, 1]], ebuf.at[b], seme[b])

    def _compute_scatter(sl_, b):
        pltpu.make_async_copy(h_hbm.at[idx5.at[sl_, 0]], hbuf.at[b],
                              semh[b]).wait()
        pltpu.make_async_copy(ctab_sh.at[idx5.at[sl_, 1]], ebuf.at[b],
                              seme[b]).wait()

        @plsc.parallel_loop(0, CHE, 1, unroll=4)
        def _row(r):
            for k in range(8):
                s2 = pl.ds(k * 16, 16)
                hbuf[b, r, s2] = jnp.maximum(hbuf[b, r, s2] + ebuf[b, r, s2],
                                             0.0)

        pltpu.async_copy(hbuf.at[b], aggr.at[idx5.at[sl_, 4]], semsc[b],
                         add=True)

    def _wait_scatter(sl_, b):
        pltpu.make_async_copy(hbuf.at[b], aggr.at[idx5.at[sl_, 4]],
                              semsc[b]).wait()

    _start_idx(0, 0)
    _start_idx(1, 1)
    _wait_idx(0, 0)
    _start_gathers(0, 0)

    @pl.loop(0, nch + 1, step=6)
    def _body(t):
        for ph in range(6):
            g = t + ph
            b = ph % 2
            sl_g = ph % 3
            sl_n = (ph + 1) % 3
            sl_p = (ph + 2) % 3

            @pl.when(jnp.logical_and(g >= 1, g <= nch))
            def _():
                _wait_scatter(sl_p, 1 - b)

            @pl.when(g + 2 < nch)
            def _():
                _start_idx(g + 2, sl_p)

            @pl.when(g + 1 < nch)
            def _():
                _wait_idx(g + 1, sl_n)
                _start_gathers(sl_n, 1 - b)

            @pl.when(g < nch)
            def _():
                _compute_scatter(sl_g, b)

    plsc.subcore_barrier()
    for j in range(RPT // ZR):
        r0 = row0 + j * ZR
        pltpu.sync_copy(aggr.at[pl.ds(r0, ZR)], out_hbm.at[c, pl.ds(r0, ZR)])


BN_BLK = 2000
NBLK = N // BN_BLK


def _ctab_body(bt_ref, a_ref, b_ref, c_ref, out_ref):
    r = jnp.dot(a_ref[...], bt_ref[0, 0], precision=_HIGH,
                preferred_element_type=jnp.float32)
    r = r + jnp.dot(b_ref[...], bt_ref[0, 1], precision=_HIGH,
                    preferred_element_type=jnp.float32)
    r = r + jnp.dot(c_ref[...], bt_ref[0, 2], precision=_HIGH,
                    preferred_element_type=jnp.float32)
    out_ref[0] = r


_ctab_call = pl.pallas_call(
    _ctab_body,
    grid=(L,),
    in_specs=[
        pl.BlockSpec((1, 3, NB, H), lambda l: (l, 0, 0, 0)),
        pl.BlockSpec((216, NB), lambda l: (0, 0)),
        pl.BlockSpec((216, NB), lambda l: (0, 0)),
        pl.BlockSpec((216, NB), lambda l: (0, 0)),
    ],
    out_specs=pl.BlockSpec((1, 216, H), lambda l: (l, 0, 0)),
    out_shape=jax.ShapeDtypeStruct((L, 216, H), jnp.float32),
)


def _mlp_body(h_ref, a_ref, eps_ref, w1_ref, b1_ref, g1_ref, bb1_ref,
              w2_ref, b2_ref, out_ref, z1s, ssum, ssq):
    p = pl.program_id(0)
    i = pl.program_id(1)

    @pl.when(p == 0)
    def _():
        zin = (1.0 + eps_ref[0, 0]) * h_ref[...] + a_ref[0] + a_ref[1]
        z1 = jnp.dot(zin, w1_ref[...], precision=_HIGH,
                     preferred_element_type=jnp.float32) + b1_ref[...]
        z1s[pl.ds(i * BN_BLK, BN_BLK), :] = z1
        s1 = jnp.sum(z1, axis=0, keepdims=True)
        s2 = jnp.sum(z1 * z1, axis=0, keepdims=True)

        @pl.when(i == 0)
        def _():
            ssum[...] = s1
            ssq[...] = s2

        @pl.when(i > 0)
        def _():
            ssum[...] = ssum[...] + s1
            ssq[...] = ssq[...] + s2

    @pl.when(p == 1)
    def _():
        mu = ssum[...] * (1.0 / N)
        var = ssq[...] * (1.0 / N) - mu * mu
        z = z1s[pl.ds(i * BN_BLK, BN_BLK), :]
        zn = g1_ref[...] * (z - mu) / jnp.sqrt(var + 1e-5) + bb1_ref[...]
        zn = jnp.maximum(zn, 0.0)
        out_ref[...] = jnp.dot(zn, w2_ref[...], precision=_HIGH,
                               preferred_element_type=jnp.float32) + b2_ref[...]


_mlp_call = pl.pallas_call(
    _mlp_body,
    grid=(2, NBLK),
    in_specs=[
        pl.BlockSpec((BN_BLK, H), lambda p, i: (i * (1 - p), 0)),
        pl.BlockSpec((NC, BN_BLK, H), lambda p, i: (0, i * (1 - p), 0)),
        pl.BlockSpec((1, 1), lambda p, i: (0, 0)),
        pl.BlockSpec((H, 2 * H), lambda p, i: (0, 0)),
        pl.BlockSpec((1, 2 * H), lambda p, i: (0, 0)),
        pl.BlockSpec((1, 2 * H), lambda p, i: (0, 0)),
        pl.BlockSpec((1, 2 * H), lambda p, i: (0, 0)),
        pl.BlockSpec((2 * H, H), lambda p, i: (0, 0)),
        pl.BlockSpec((1, H), lambda p, i: (0, 0)),
    ],
    out_specs=pl.BlockSpec((BN_BLK, H), lambda p, i: (i, 0)),
    out_shape=jax.ShapeDtypeStruct((N, H), jnp.float32),
    scratch_shapes=[
        pltpu.VMEM((N, 2 * H), jnp.float32),
        pltpu.VMEM((1, 2 * H), jnp.float32),
        pltpu.VMEM((1, 2 * H), jnp.float32),
    ],
)


def _pool_body(h_ref, batch_ref, wout_ref, bout_ref, out_ref, acc_ref):
    i = pl.program_id(0)
    b = batch_ref[0, 0]
    oh = (b[:, None] == lax.broadcasted_iota(jnp.int32, (1, G), 1)
          ).astype(jnp.float32)
    part = lax.dot_general(oh, h_ref[...], (((0,), (0,)), ((), ())),
                           precision=_HIGH, preferred_element_type=jnp.float32)

    @pl.when(i == 0)
    def _():
        acc_ref[...] = part

    @pl.when(i > 0)
    def _():
        acc_ref[...] = acc_ref[...] + part

    @pl.when(i == NBLK - 1)
    def _():
        out_ref[...] = jnp.dot(acc_ref[...], wout_ref[...], precision=_HIGH,
                               preferred_element_type=jnp.float32) + bout_ref[...]


_pool_call = pl.pallas_call(
    _pool_body,
    grid=(NBLK,),
    in_specs=[
        pl.BlockSpec((BN_BLK, H), lambda i: (i, 0)),
        pl.BlockSpec((1, 1, BN_BLK), lambda i: (i, 0, 0)),
        pl.BlockSpec((H, C), lambda i: (0, 0)),
        pl.BlockSpec((1, C), lambda i: (0, 0)),
    ],
    out_specs=pl.BlockSpec((G, C), lambda i: (0, 0)),
    out_shape=jax.ShapeDtypeStruct((G, C), jnp.float32),
    scratch_shapes=[pltpu.VMEM((G, H), jnp.float32)],
)


def kernel(x, edge_index, edge_attr, batch, atom_tables, bond_tables, eps,
           W1, b1, g1, bb1, W2, b2, Wout, bout):
    x = x.astype(jnp.int32)
    ei = edge_index.astype(jnp.int32)
    ea = edge_attr.astype(jnp.int32)
    batch = batch.astype(jnp.int32)

    xt = jnp.pad(x.T, ((0, 0), (0, NPAD - N)))
    xc = xt.reshape(9, NCHN_TOT, CHN).transpose(1, 0, 2)
    atab = atom_tables.reshape(9 * NA, H)
    h = _atom_encode_sc(xc, atab)[:N]

    cid = jnp.arange(216)
    oh_a = jax.nn.one_hot(cid // 36, NB, dtype=jnp.float32)
    oh_b = jax.nn.one_hot((cid // 6) % NB, NB, dtype=jnp.float32)
    oh_c = jax.nn.one_hot(cid % NB, NB, dtype=jnp.float32)
    ctab = _ctab_call(bond_tables, oh_a, oh_b, oh_c)

    einds = jnp.stack([ei[0], ea[:, 0], ea[:, 1], ea[:, 2], ei[1]], axis=0)
    einds = einds.reshape(5, NCHT, CHE).transpose(1, 0, 2)

    batch3 = batch.reshape(NBLK, 1, BN_BLK)

    for l in range(L):
        aggr2 = _edge_pass_sc(h, einds, ctab[l])
        h = _mlp_call(h, aggr2, eps[l].reshape(1, 1), W1[l],
                      b1[l].reshape(1, 2 * H), g1[l].reshape(1, 2 * H),
                      bb1[l].reshape(1, 2 * H), W2[l], b2[l].reshape(1, H))

    return _pool_call(h, batch3, Wout, bout.reshape(1, C))
